# Initial kernel scaffold; baseline (speedup 1.0000x reference)
import jax
import jax.numpy as jnp
from jax.experimental import pallas as pl


def kernel(x_seq, h0, U_c, W_c, b_c, U_a, W_a, b_a, U_h, b_h):
    raise NotImplementedError("write your pallas kernel here")


# speedup vs baseline: 5.8964x; 5.8964x over previous
import functools

import jax
import jax.numpy as jnp
from jax.experimental import pallas as pl
from jax.experimental.pallas import tpu as pltpu

_CT = 8


def _nbrc_body(CT, H, x_ref, h0_ref, u_ref, w_ref, b_ref, y_ref, hf_ref,
               h_s, xp_s):
    B = h0_ref.shape[0]
    t0 = pl.program_id(0)

    @pl.when(t0 == 0)
    def _():
        h_s[...] = h0_ref[...]

    xp_s[...] = (
        jnp.dot(x_ref[...], u_ref[...], preferred_element_type=jnp.float32)
        + b_ref[...]
    )

    h = h_s[...]
    for t in range(CT):
        r = slice(t * B, (t + 1) * B)
        ca = jnp.dot(h, w_ref[...], preferred_element_type=jnp.float32)
        c = jax.nn.sigmoid(xp_s[r, :H] + ca[:, :H])
        a = 1.0 + jnp.tanh(xp_s[r, H:2 * H] + ca[:, H:])
        hn = c * h + (1.0 - c) * jnp.tanh(xp_s[r, 2 * H:] + a * h)
        y_ref[r, :] = hn
        h = hn
    h_s[...] = h

    @pl.when(t0 == pl.num_programs(0) - 1)
    def _():
        hf_ref[...] = h


def kernel(x_seq, h0, U_c, W_c, b_c, U_a, W_a, b_a, U_h, b_h):
    T, B, I = x_seq.shape
    H = h0.shape[1]
    CT = _CT

    x2 = x_seq.reshape(T * B, I)
    Ut = jnp.concatenate([U_c.T, U_a.T, U_h.T], axis=1)
    Wt = jnp.concatenate([W_c.T, W_a.T], axis=1)
    bb = jnp.concatenate([b_c, b_a, b_h]).reshape(1, 3 * H)

    y2, hf = pl.pallas_call(
        functools.partial(_nbrc_body, CT, H),
        grid=(T // CT,),
        in_specs=[
            pl.BlockSpec((CT * B, I), lambda t: (t, 0)),
            pl.BlockSpec((B, H), lambda t: (0, 0)),
            pl.BlockSpec((I, 3 * H), lambda t: (0, 0)),
            pl.BlockSpec((H, 2 * H), lambda t: (0, 0)),
            pl.BlockSpec((1, 3 * H), lambda t: (0, 0)),
        ],
        out_specs=[
            pl.BlockSpec((CT * B, H), lambda t: (t, 0)),
            pl.BlockSpec((B, H), lambda t: (0, 0)),
        ],
        out_shape=[
            jax.ShapeDtypeStruct((T * B, H), jnp.float32),
            jax.ShapeDtypeStruct((B, H), jnp.float32),
        ],
        scratch_shapes=[
            pltpu.VMEM((B, H), jnp.float32),
            pltpu.VMEM((CT * B, 3 * H), jnp.float32),
        ],
        compiler_params=pltpu.CompilerParams(
            dimension_semantics=("arbitrary",),
            vmem_limit_bytes=50 * 1024 * 1024,
        ),
        name="nbrc_scan",
    )(x2, h0, Ut, Wt, bb)

    return (y2.reshape(T, B, H), (hf,))
